# baseline (speedup 1.0000x reference)
import functools

import jax
import jax.numpy as jnp
import numpy as np
from jax import lax
from jax.experimental import pallas as pl
from jax.experimental.pallas import tpu as pltpu
from jax.experimental.pallas import tpu_sc as plsc

N = 10000
E = 320000
D_IN = 256
H = 128

NC = 2
NS = 16
K = 128
CHT = 160
QP = CHT // 4
DEG_CH = CHT // NC
E_PAD = NS * CHT * K
PAD_COL = 10100
N_PAD = 10240
SLICE = N_PAD // NS

f32 = jnp.float32


def _sc_mesh():
    return plsc.VectorSubcoreMesh(core_axis_name="c", subcore_axis_name="s")


def _fill(ref, value):
    vec = jnp.full((16,), value, f32)

    def body(i, _):
        for t in range(H // 16):
            ref[i, pl.ds(t * 16, 16)] = vec
        return 0

    lax.fori_loop(0, K, body, 0)


def _make_deg_kernel():
    @functools.partial(
        pl.kernel,
        mesh=_sc_mesh(),
        out_type=jax.ShapeDtypeStruct((NC, N_PAD, H), f32),
        scratch_types=[
            pltpu.VMEM((DEG_CH, K), jnp.int32),
            pltpu.VMEM((K, H), f32),
            pltpu.VMEM_SHARED((N_PAD, H), f32),
        ],
    )
    def deg_kernel(col_hbm, out_hbm, col_v, ones_v, acc_sh):
        c = lax.axis_index("c")
        s = lax.axis_index("s")
        pltpu.sync_copy(col_hbm.at[s, pl.ds(c * DEG_CH, DEG_CH)], col_v)
        _fill(ones_v, 0.0)
        for b in range(SLICE // K):
            pltpu.sync_copy(ones_v, acc_sh.at[pl.ds(s * SLICE + b * K, K)])
        _fill(ones_v, 1.0)
        plsc.subcore_barrier()

        def chunk_body(j, _):
            pltpu.sync_copy(ones_v, acc_sh.at[col_v.at[j]], add=True)
            return 0

        lax.fori_loop(0, DEG_CH, chunk_body, 0)
        plsc.subcore_barrier()
        pltpu.sync_copy(acc_sh.at[pl.ds(s * SLICE, SLICE)],
                        out_hbm.at[c, pl.ds(s * SLICE, SLICE)])

    return deg_kernel


def _make_scatter_kernel():
    @functools.partial(
        pl.kernel,
        mesh=_sc_mesh(),
        out_type=jax.ShapeDtypeStruct((N_PAD, H), f32),
        scratch_types=[
            pltpu.VMEM((QP, K), jnp.int32),
            pltpu.VMEM((QP, K), jnp.int32),
            pltpu.VMEM((K, H), f32),
            pltpu.VMEM((K, H), f32),
            pltpu.VMEM_SHARED((N_PAD, H), f32),
            pltpu.SemaphoreType.DMA,
            pltpu.SemaphoreType.DMA,
        ],
    )
    def scatter_kernel(row_hbm, col_hbm, a_hbm, out_hbm,
                       row_v, col_v, rows_v0, rows_v1, acc_sh, sem0, sem1):
        c = lax.axis_index("c")
        s = lax.axis_index("s")

        @pl.when(c == 0)
        def _core0():
            _fill(rows_v0, 0.0)
            for b in range(SLICE // K):
                pltpu.sync_copy(rows_v0, acc_sh.at[pl.ds(s * SLICE + b * K, K)])
            plsc.subcore_barrier()

            for p in range(4):
                pltpu.sync_copy(row_hbm.at[s, pl.ds(p * QP, QP)], row_v)
                pltpu.sync_copy(col_hbm.at[s, pl.ds(p * QP, QP)], col_v)
                pltpu.async_copy(a_hbm.at[row_v.at[0]], rows_v0, sem0)

                def chunk_body(g, _):
                    j0 = 2 * g
                    pltpu.make_async_copy(a_hbm.at[row_v.at[j0]], rows_v0, sem0).wait()
                    pltpu.async_copy(a_hbm.at[row_v.at[j0 + 1]], rows_v1, sem1)
                    pltpu.sync_copy(rows_v0, acc_sh.at[col_v.at[j0]], add=True)
                    pltpu.make_async_copy(a_hbm.at[row_v.at[j0 + 1]], rows_v1, sem1).wait()

                    @pl.when(g + 1 < QP // 2)
                    def _prefetch():
                        pltpu.async_copy(a_hbm.at[row_v.at[j0 + 2]], rows_v0, sem0)

                    pltpu.sync_copy(rows_v1, acc_sh.at[col_v.at[j0 + 1]], add=True)
                    return 0

                lax.fori_loop(0, QP // 2, chunk_body, 0)

            plsc.subcore_barrier()
            pltpu.sync_copy(acc_sh.at[pl.ds(s * SLICE, SLICE)],
                            out_hbm.at[pl.ds(s * SLICE, SLICE)])

    return scatter_kernel


_deg_sc = _make_deg_kernel()
_scatter_sc = _make_scatter_kernel()


_R = 1024
_RC = 1000


def _pre_body(x_ref, wpt_ref, bp_ref, wat_ref, wbt_ref, mb_ref, degp_ref,
              h_ref, a_ref, b_ref, deg_ref):
    h = jnp.dot(x_ref[...], wpt_ref[...], preferred_element_type=f32)
    h = h + bp_ref[...]
    h_ref[...] = h
    a_ref[...] = jnp.dot(h, wat_ref[...], preferred_element_type=f32)
    b_ref[...] = jnp.dot(h, wbt_ref[...], preferred_element_type=f32) + mb_ref[...]
    deg_ref[...] = degp_ref[0][:, 0] + degp_ref[1][:, 0]


def _pre_tc(x, wpt, bp, wat, wbt, mb, deg_parts):
    return pl.pallas_call(
        _pre_body,
        grid=(10,),
        in_specs=[
            pl.BlockSpec((_R, D_IN), lambda r: (r, 0)),
            pl.BlockSpec((D_IN, H), lambda r: (0, 0)),
            pl.BlockSpec((1, H), lambda r: (0, 0)),
            pl.BlockSpec((H, H), lambda r: (0, 0)),
            pl.BlockSpec((H, H), lambda r: (0, 0)),
            pl.BlockSpec((1, H), lambda r: (0, 0)),
            pl.BlockSpec((NC, _R, H), lambda r: (0, r, 0)),
        ],
        out_specs=[
            pl.BlockSpec((_R, H), lambda r: (r, 0)),
            pl.BlockSpec((_R, H), lambda r: (r, 0)),
            pl.BlockSpec((_R, H), lambda r: (r, 0)),
            pl.BlockSpec((_R,), lambda r: (r,)),
        ],
        out_shape=[
            jax.ShapeDtypeStruct((N_PAD, H), f32),
            jax.ShapeDtypeStruct((N_PAD, H), f32),
            jax.ShapeDtypeStruct((N_PAD, H), f32),
            jax.ShapeDtypeStruct((N_PAD,), f32),
        ],
    )(x, wpt, bp, wat, wbt, mb, deg_parts)


def _gru_core(h, s_ref, b_ref, deg_ref, wih_ref, whh_ref, bih_ref, bhh_ref):
    aggr = s_ref[...] + deg_ref[...][:, None] * b_ref[...]
    gi = jnp.dot(aggr, wih_ref[...], preferred_element_type=f32) + bih_ref[...]
    gh = jnp.dot(h, whh_ref[...], preferred_element_type=f32) + bhh_ref[...]
    r = jax.nn.sigmoid(gi[:, :H] + gh[:, :H])
    z = jax.nn.sigmoid(gi[:, H:2 * H] + gh[:, H:2 * H])
    n = jnp.tanh(gi[:, 2 * H:] + r * gh[:, 2 * H:])
    return (1.0 - z) * n + z * h


def _layer_body(h_ref, s_ref, b_ref, deg_ref, wih_ref, whh_ref,
                bih_ref, bhh_ref, wat_ref, wbt_ref, mb_ref,
                hn_ref, an_ref, bn_ref):
    hn = _gru_core(h_ref[...], s_ref, b_ref, deg_ref,
                   wih_ref, whh_ref, bih_ref, bhh_ref)
    hn_ref[...] = hn
    an_ref[...] = jnp.dot(hn, wat_ref[...], preferred_element_type=f32)
    bn_ref[...] = jnp.dot(hn, wbt_ref[...], preferred_element_type=f32) + mb_ref[...]


def _final_body(h_ref, s_ref, b_ref, deg_ref, wih_ref, whh_ref,
                bih_ref, bhh_ref, hn_ref):
    hn_ref[...] = _gru_core(h_ref[...], s_ref, b_ref, deg_ref,
                            wih_ref, whh_ref, bih_ref, bhh_ref)


_LAYER_IN_SPECS = [
    pl.BlockSpec((_R, H), lambda r: (r, 0)),
    pl.BlockSpec((_R, H), lambda r: (r, 0)),
    pl.BlockSpec((_R, H), lambda r: (r, 0)),
    pl.BlockSpec((_R,), lambda r: (r,)),
    pl.BlockSpec((H, 3 * H), lambda r: (0, 0)),
    pl.BlockSpec((H, 3 * H), lambda r: (0, 0)),
    pl.BlockSpec((1, 3 * H), lambda r: (0, 0)),
    pl.BlockSpec((1, 3 * H), lambda r: (0, 0)),
]


def _layer_tc(h, s_parts, b_cur, deg, wih_t, whh_t, bih, bhh, wat, wbt, mb):
    return pl.pallas_call(
        _layer_body,
        grid=(10,),
        in_specs=_LAYER_IN_SPECS + [
            pl.BlockSpec((H, H), lambda r: (0, 0)),
            pl.BlockSpec((H, H), lambda r: (0, 0)),
            pl.BlockSpec((1, H), lambda r: (0, 0)),
        ],
        out_specs=[
            pl.BlockSpec((_R, H), lambda r: (r, 0)),
            pl.BlockSpec((_R, H), lambda r: (r, 0)),
            pl.BlockSpec((_R, H), lambda r: (r, 0)),
        ],
        out_shape=[
            jax.ShapeDtypeStruct((N_PAD, H), f32),
            jax.ShapeDtypeStruct((N_PAD, H), f32),
            jax.ShapeDtypeStruct((N_PAD, H), f32),
        ],
    )(h, s_parts, b_cur, deg, wih_t, whh_t, bih, bhh, wat, wbt, mb)


def _final_tc(h, s_parts, b_cur, deg, wih_t, whh_t, bih, bhh):
    return pl.pallas_call(
        _final_body,
        grid=(10,),
        in_specs=_LAYER_IN_SPECS,
        out_specs=pl.BlockSpec((_R, H), lambda r: (r, 0)),
        out_shape=jax.ShapeDtypeStruct((N_PAD, H), f32),
    )(h, s_parts, b_cur, deg, wih_t, whh_t, bih, bhh)


def _cls_body(ht_ref, hb_ref, w1a_ref, w1b_ref, b1_ref, w2_ref, b2_ref, o_ref):
    hid = jnp.dot(ht_ref[...], w1a_ref[...], preferred_element_type=f32)
    hid = hid + jnp.dot(hb_ref[...], w1b_ref[...], preferred_element_type=f32)
    hid = jax.nn.relu(hid + b1_ref[...])
    o_ref[...] = jnp.dot(hid, w2_ref[...], preferred_element_type=f32) + b2_ref[...]


def _cls_tc(h, w1a, w1b, b1, w2p, b2p):
    nb = N // 2 // _RC
    return pl.pallas_call(
        _cls_body,
        grid=(nb,),
        in_specs=[
            pl.BlockSpec((_RC, H), lambda r: (r, 0)),
            pl.BlockSpec((_RC, H), lambda r: (r + nb, 0)),
            pl.BlockSpec((H, H), lambda r: (0, 0)),
            pl.BlockSpec((H, H), lambda r: (0, 0)),
            pl.BlockSpec((1, H), lambda r: (0, 0)),
            pl.BlockSpec((H, H), lambda r: (0, 0)),
            pl.BlockSpec((1, H), lambda r: (0, 0)),
        ],
        out_specs=pl.BlockSpec((_RC, H), lambda r: (r, 0)),
        out_shape=jax.ShapeDtypeStruct((N // 2, H), f32),
    )(h, h, w1a, w1b, b1, w2p, b2p)


def kernel(x, edge_index, W_proj, b_proj,
           msg_W0, msg_b0, gru_Wih0, gru_Whh0, gru_bih0, gru_bhh0,
           msg_W1, msg_b1, gru_Wih1, gru_Whh1, gru_bih1, gru_bhh1,
           msg_W2, msg_b2, gru_Wih2, gru_Whh2, gru_bih2, gru_bhh2,
           cls_W1, cls_b1, cls_W2, cls_b2):
    msg_W = [msg_W0, msg_W1, msg_W2]
    msg_b = [msg_b0, msg_b1, msg_b2]
    gru = [(gru_Wih0, gru_Whh0, gru_bih0, gru_bhh0),
           (gru_Wih1, gru_Whh1, gru_bih1, gru_bhh1),
           (gru_Wih2, gru_Whh2, gru_bih2, gru_bhh2)]

    pad = E_PAD - E
    row_t = jnp.concatenate(
        [edge_index[0], jnp.zeros((pad,), jnp.int32)]).reshape(NS, CHT, K)
    col_t = jnp.concatenate(
        [edge_index[1], jnp.full((pad,), PAD_COL, jnp.int32)]).reshape(NS, CHT, K)

    wat = [w[:, :H].T for w in msg_W]
    wbt = [w[:, H:].T for w in msg_W]
    mb = [b.reshape(1, H) for b in msg_b]

    deg_parts = _deg_sc(col_t)
    h, a_cur, b_cur, deg = _pre_tc(
        x, W_proj.T, b_proj.reshape(1, H), wat[0], wbt[0], mb[0], deg_parts)

    for i in range(3):
        s_parts = _scatter_sc(row_t, col_t, a_cur)
        wih_t = gru[i][0].T
        whh_t = gru[i][1].T
        bih = gru[i][2].reshape(1, 3 * H)
        bhh = gru[i][3].reshape(1, 3 * H)
        if i < 2:
            h, a_cur, b_cur = _layer_tc(h, s_parts, b_cur, deg, wih_t, whh_t,
                                        bih, bhh, wat[i + 1], wbt[i + 1], mb[i + 1])
        else:
            h = _final_tc(h, s_parts, b_cur, deg, wih_t, whh_t, bih, bhh)

    w2p = jnp.zeros((H, H), f32).at[:, :2].set(cls_W2.T)
    b2p = jnp.zeros((1, H), f32).at[0, :2].set(cls_b2)
    out_pad = _cls_tc(h, cls_W1[:, :H].T, cls_W1[:, H:].T,
                      cls_b1.reshape(1, H), w2p, b2p)
    return out_pad[:, :2]


# speedup vs baseline: 3.9530x; 3.9530x over previous
import functools

import jax
import jax.numpy as jnp
import numpy as np
from jax import lax
from jax.experimental import pallas as pl
from jax.experimental.pallas import tpu as pltpu
from jax.experimental.pallas import tpu_sc as plsc

N = 10000
E = 320000
D_IN = 256
H = 128

NC = 2
NS = 16
K = 128
CHT = 160
CH = CHT // NC
HP = CH // 2
DEG_CH = CH
E_PAD = NS * CHT * K
PAD_COL = 10100
N_PAD = 10240
SLICE = N_PAD // NS

f32 = jnp.float32


def _sc_mesh():
    return plsc.VectorSubcoreMesh(core_axis_name="c", subcore_axis_name="s")


def _fill(ref, value):
    vec = jnp.full((16,), value, f32)

    def body(i, _):
        for t in range(H // 16):
            ref[i, pl.ds(t * 16, 16)] = vec
        return 0

    lax.fori_loop(0, K, body, 0)


def _make_deg_kernel():
    @functools.partial(
        pl.kernel,
        mesh=_sc_mesh(),
        out_type=jax.ShapeDtypeStruct((NC, N_PAD, H), f32),
        scratch_types=[
            pltpu.VMEM((DEG_CH, K), jnp.int32),
            pltpu.VMEM((K, H), f32),
            pltpu.VMEM_SHARED((N_PAD, H), f32),
        ],
    )
    def deg_kernel(col_hbm, out_hbm, col_v, ones_v, acc_sh):
        c = lax.axis_index("c")
        s = lax.axis_index("s")
        pltpu.sync_copy(col_hbm.at[s, pl.ds(c * DEG_CH, DEG_CH)], col_v)
        _fill(ones_v, 0.0)
        for b in range(SLICE // K):
            pltpu.sync_copy(ones_v, acc_sh.at[pl.ds(s * SLICE + b * K, K)])
        _fill(ones_v, 1.0)
        plsc.subcore_barrier()

        def chunk_body(j, _):
            pltpu.sync_copy(ones_v, acc_sh.at[col_v.at[j]], add=True)
            return 0

        lax.fori_loop(0, DEG_CH, chunk_body, 0)
        plsc.subcore_barrier()
        pltpu.sync_copy(acc_sh.at[pl.ds(s * SLICE, SLICE)],
                        out_hbm.at[c, pl.ds(s * SLICE, SLICE)])

    return deg_kernel


def _make_scatter_kernel():
    @functools.partial(
        pl.kernel,
        mesh=_sc_mesh(),
        out_type=jax.ShapeDtypeStruct((NC, N_PAD, H), f32),
        scratch_types=[
            pltpu.VMEM((HP, K), jnp.int32),
            pltpu.VMEM((HP, K), jnp.int32),
            pltpu.VMEM((K, H), f32),
            pltpu.VMEM((K, H), f32),
            pltpu.VMEM_SHARED((N_PAD, H), f32),
            pltpu.SemaphoreType.DMA,
            pltpu.SemaphoreType.DMA,
        ],
    )
    def scatter_kernel(row_hbm, col_hbm, a_hbm, out_hbm,
                       row_v, col_v, rows_v0, rows_v1, acc_sh, sem0, sem1):
        c = lax.axis_index("c")
        s = lax.axis_index("s")

        _fill(rows_v0, 0.0)
        for b in range(SLICE // K):
            pltpu.sync_copy(rows_v0, acc_sh.at[pl.ds(s * SLICE + b * K, K)])
        plsc.subcore_barrier()

        for p in range(2):
            off = c * CH + p * HP
            pltpu.sync_copy(row_hbm.at[s, pl.ds(off, HP)], row_v)
            pltpu.sync_copy(col_hbm.at[s, pl.ds(off, HP)], col_v)
            pltpu.async_copy(a_hbm.at[row_v.at[0]], rows_v0, sem0)

            def chunk_body(g, _):
                j0 = 2 * g
                pltpu.make_async_copy(a_hbm.at[row_v.at[j0]], rows_v0, sem0).wait()
                pltpu.async_copy(a_hbm.at[row_v.at[j0 + 1]], rows_v1, sem1)
                pltpu.sync_copy(rows_v0, acc_sh.at[col_v.at[j0]], add=True)
                pltpu.make_async_copy(a_hbm.at[row_v.at[j0 + 1]], rows_v1, sem1).wait()

                @pl.when(g + 1 < HP // 2)
                def _prefetch():
                    pltpu.async_copy(a_hbm.at[row_v.at[j0 + 2]], rows_v0, sem0)

                pltpu.sync_copy(rows_v1, acc_sh.at[col_v.at[j0 + 1]], add=True)
                return 0

            lax.fori_loop(0, HP // 2, chunk_body, 0)

        plsc.subcore_barrier()
        pltpu.sync_copy(acc_sh.at[pl.ds(s * SLICE, SLICE)],
                        out_hbm.at[c, pl.ds(s * SLICE, SLICE)])

    return scatter_kernel


_deg_sc = _make_deg_kernel()
_scatter_sc = _make_scatter_kernel()


_R = 1024
_RC = 1000


def _pre_body(x_ref, wpt_ref, bp_ref, wat_ref, wbt_ref, mb_ref, degp_ref,
              h_ref, a_ref, b_ref, deg_ref):
    h = jnp.dot(x_ref[...], wpt_ref[...], preferred_element_type=f32)
    h = h + bp_ref[...]
    h_ref[...] = h
    a_ref[...] = jnp.dot(h, wat_ref[...], preferred_element_type=f32)
    b_ref[...] = jnp.dot(h, wbt_ref[...], preferred_element_type=f32) + mb_ref[...]
    deg_ref[...] = degp_ref[0][:, 0] + degp_ref[1][:, 0]


def _pre_tc(x, wpt, bp, wat, wbt, mb, deg_parts):
    return pl.pallas_call(
        _pre_body,
        grid=(10,),
        in_specs=[
            pl.BlockSpec((_R, D_IN), lambda r: (r, 0)),
            pl.BlockSpec((D_IN, H), lambda r: (0, 0)),
            pl.BlockSpec((1, H), lambda r: (0, 0)),
            pl.BlockSpec((H, H), lambda r: (0, 0)),
            pl.BlockSpec((H, H), lambda r: (0, 0)),
            pl.BlockSpec((1, H), lambda r: (0, 0)),
            pl.BlockSpec((NC, _R, H), lambda r: (0, r, 0)),
        ],
        out_specs=[
            pl.BlockSpec((_R, H), lambda r: (r, 0)),
            pl.BlockSpec((_R, H), lambda r: (r, 0)),
            pl.BlockSpec((_R, H), lambda r: (r, 0)),
            pl.BlockSpec((_R,), lambda r: (r,)),
        ],
        out_shape=[
            jax.ShapeDtypeStruct((N_PAD, H), f32),
            jax.ShapeDtypeStruct((N_PAD, H), f32),
            jax.ShapeDtypeStruct((N_PAD, H), f32),
            jax.ShapeDtypeStruct((N_PAD,), f32),
        ],
    )(x, wpt, bp, wat, wbt, mb, deg_parts)


def _gru_core(h, s_ref, b_ref, deg_ref, wih_ref, whh_ref, bih_ref, bhh_ref):
    aggr = s_ref[0] + s_ref[1] + deg_ref[...][:, None] * b_ref[...]
    gi = jnp.dot(aggr, wih_ref[...], preferred_element_type=f32) + bih_ref[...]
    gh = jnp.dot(h, whh_ref[...], preferred_element_type=f32) + bhh_ref[...]
    r = jax.nn.sigmoid(gi[:, :H] + gh[:, :H])
    z = jax.nn.sigmoid(gi[:, H:2 * H] + gh[:, H:2 * H])
    n = jnp.tanh(gi[:, 2 * H:] + r * gh[:, 2 * H:])
    return (1.0 - z) * n + z * h


def _layer_body(h_ref, s_ref, b_ref, deg_ref, wih_ref, whh_ref,
                bih_ref, bhh_ref, wat_ref, wbt_ref, mb_ref,
                hn_ref, an_ref, bn_ref):
    hn = _gru_core(h_ref[...], s_ref, b_ref, deg_ref,
                   wih_ref, whh_ref, bih_ref, bhh_ref)
    hn_ref[...] = hn
    an_ref[...] = jnp.dot(hn, wat_ref[...], preferred_element_type=f32)
    bn_ref[...] = jnp.dot(hn, wbt_ref[...], preferred_element_type=f32) + mb_ref[...]


def _final_body(h_ref, s_ref, b_ref, deg_ref, wih_ref, whh_ref,
                bih_ref, bhh_ref, hn_ref):
    hn_ref[...] = _gru_core(h_ref[...], s_ref, b_ref, deg_ref,
                            wih_ref, whh_ref, bih_ref, bhh_ref)


_LAYER_IN_SPECS = [
    pl.BlockSpec((_R, H), lambda r: (r, 0)),
    pl.BlockSpec((NC, _R, H), lambda r: (0, r, 0)),
    pl.BlockSpec((_R, H), lambda r: (r, 0)),
    pl.BlockSpec((_R,), lambda r: (r,)),
    pl.BlockSpec((H, 3 * H), lambda r: (0, 0)),
    pl.BlockSpec((H, 3 * H), lambda r: (0, 0)),
    pl.BlockSpec((1, 3 * H), lambda r: (0, 0)),
    pl.BlockSpec((1, 3 * H), lambda r: (0, 0)),
]


def _layer_tc(h, s_parts, b_cur, deg, wih_t, whh_t, bih, bhh, wat, wbt, mb):
    return pl.pallas_call(
        _layer_body,
        grid=(10,),
        in_specs=_LAYER_IN_SPECS + [
            pl.BlockSpec((H, H), lambda r: (0, 0)),
            pl.BlockSpec((H, H), lambda r: (0, 0)),
            pl.BlockSpec((1, H), lambda r: (0, 0)),
        ],
        out_specs=[
            pl.BlockSpec((_R, H), lambda r: (r, 0)),
            pl.BlockSpec((_R, H), lambda r: (r, 0)),
            pl.BlockSpec((_R, H), lambda r: (r, 0)),
        ],
        out_shape=[
            jax.ShapeDtypeStruct((N_PAD, H), f32),
            jax.ShapeDtypeStruct((N_PAD, H), f32),
            jax.ShapeDtypeStruct((N_PAD, H), f32),
        ],
    )(h, s_parts, b_cur, deg, wih_t, whh_t, bih, bhh, wat, wbt, mb)


def _final_tc(h, s_parts, b_cur, deg, wih_t, whh_t, bih, bhh):
    return pl.pallas_call(
        _final_body,
        grid=(10,),
        in_specs=_LAYER_IN_SPECS,
        out_specs=pl.BlockSpec((_R, H), lambda r: (r, 0)),
        out_shape=jax.ShapeDtypeStruct((N_PAD, H), f32),
    )(h, s_parts, b_cur, deg, wih_t, whh_t, bih, bhh)


def _cls_body(ht_ref, hb_ref, w1a_ref, w1b_ref, b1_ref, w2_ref, b2_ref, o_ref):
    hid = jnp.dot(ht_ref[...], w1a_ref[...], preferred_element_type=f32)
    hid = hid + jnp.dot(hb_ref[...], w1b_ref[...], preferred_element_type=f32)
    hid = jax.nn.relu(hid + b1_ref[...])
    o_ref[...] = jnp.dot(hid, w2_ref[...], preferred_element_type=f32) + b2_ref[...]


def _cls_tc(h, w1a, w1b, b1, w2p, b2p):
    nb = N // 2 // _RC
    return pl.pallas_call(
        _cls_body,
        grid=(nb,),
        in_specs=[
            pl.BlockSpec((_RC, H), lambda r: (r, 0)),
            pl.BlockSpec((_RC, H), lambda r: (r + nb, 0)),
            pl.BlockSpec((H, H), lambda r: (0, 0)),
            pl.BlockSpec((H, H), lambda r: (0, 0)),
            pl.BlockSpec((1, H), lambda r: (0, 0)),
            pl.BlockSpec((H, H), lambda r: (0, 0)),
            pl.BlockSpec((1, H), lambda r: (0, 0)),
        ],
        out_specs=pl.BlockSpec((_RC, H), lambda r: (r, 0)),
        out_shape=jax.ShapeDtypeStruct((N // 2, H), f32),
    )(h, h, w1a, w1b, b1, w2p, b2p)


def kernel(x, edge_index, W_proj, b_proj,
           msg_W0, msg_b0, gru_Wih0, gru_Whh0, gru_bih0, gru_bhh0,
           msg_W1, msg_b1, gru_Wih1, gru_Whh1, gru_bih1, gru_bhh1,
           msg_W2, msg_b2, gru_Wih2, gru_Whh2, gru_bih2, gru_bhh2,
           cls_W1, cls_b1, cls_W2, cls_b2):
    msg_W = [msg_W0, msg_W1, msg_W2]
    msg_b = [msg_b0, msg_b1, msg_b2]
    gru = [(gru_Wih0, gru_Whh0, gru_bih0, gru_bhh0),
           (gru_Wih1, gru_Whh1, gru_bih1, gru_bhh1),
           (gru_Wih2, gru_Whh2, gru_bih2, gru_bhh2)]

    pad = E_PAD - E
    pad_iota = jnp.arange(pad, dtype=jnp.int32)
    row_t = jnp.concatenate(
        [edge_index[0], pad_iota % N]).reshape(NS, CHT, K)
    col_t = jnp.concatenate(
        [edge_index[1], N + pad_iota % K]).reshape(NS, CHT, K)

    wat = [w[:, :H].T for w in msg_W]
    wbt = [w[:, H:].T for w in msg_W]
    mb = [b.reshape(1, H) for b in msg_b]

    deg_parts = _deg_sc(col_t)
    h, a_cur, b_cur, deg = _pre_tc(
        x, W_proj.T, b_proj.reshape(1, H), wat[0], wbt[0], mb[0], deg_parts)

    for i in range(3):
        s_parts = _scatter_sc(row_t, col_t, a_cur)
        wih_t = gru[i][0].T
        whh_t = gru[i][1].T
        bih = gru[i][2].reshape(1, 3 * H)
        bhh = gru[i][3].reshape(1, 3 * H)
        if i < 2:
            h, a_cur, b_cur = _layer_tc(h, s_parts, b_cur, deg, wih_t, whh_t,
                                        bih, bhh, wat[i + 1], wbt[i + 1], mb[i + 1])
        else:
            h = _final_tc(h, s_parts, b_cur, deg, wih_t, whh_t, bih, bhh)

    w2p = jnp.zeros((H, H), f32).at[:, :2].set(cls_W2.T)
    b2p = jnp.zeros((1, H), f32).at[0, :2].set(cls_b2)
    out_pad = _cls_tc(h, cls_W1[:, :H].T, cls_W1[:, H:].T,
                      cls_b1.reshape(1, H), w2p, b2p)
    return out_pad[:, :2]
